# baseline (speedup 1.0000x reference)
import functools

import jax
import jax.numpy as jnp
from jax import lax
from jax.experimental import pallas as pl
from jax.experimental.pallas import tpu as pltpu
from jax.experimental.pallas import tpu_sc as plsc

N_NODES = 10000
N_PAIRS = 320000
HIDDEN = 128
FH = 64
N_CLASS = 40
F_PAD = 48
FC = 24

NC, NS = 2, 16
CHUNK = 128
NB = 4
NPP = 327680
NROWS = NPP // CHUNK
CHT = NROWS // NS
R_ACC = 10016
RT = R_ACC // NS
RK, RTAIL = 64, 50
DUMMY = 10000


def _fill(ref, rows, width, value):
    def row(i, _):
        for k in range(width // 16):
            ref[i, pl.ds(16 * k, 16)] = jnp.full((16,), value, jnp.float32)
        return _
    lax.fori_loop(0, rows, row, None)


def _make_seg(F, with_deg):
    mesh = plsc.VectorSubcoreMesh(core_axis_name="c", subcore_axis_name="s")
    out_type = [jax.ShapeDtypeStruct((NC, R_ACC, F), jnp.float32)]
    if with_deg:
        out_type.append(jax.ShapeDtypeStruct((NC, R_ACC, 16), jnp.float32))
    scratch = [
        pltpu.VMEM((CHT, CHUNK), jnp.int32),
        pltpu.VMEM((CHT, CHUNK), jnp.int32),
    ]
    scratch += [pltpu.VMEM((CHUNK, F), jnp.float32) for _ in range(NB)]
    scratch.append(pltpu.VMEM((RK, F), jnp.float32))
    if with_deg:
        scratch += [
            pltpu.VMEM((CHUNK, 16), jnp.float32),
            pltpu.VMEM((RK, 16), jnp.float32),
        ]
    scratch.append(pltpu.VMEM_SHARED((R_ACC, F), jnp.float32))
    if with_deg:
        scratch.append(pltpu.VMEM_SHARED((R_ACC, 16), jnp.float32))
    scratch += [pltpu.SemaphoreType.DMA] * (2 * NB)
    if with_deg:
        scratch += [pltpu.SemaphoreType.DMA] * NB

    @functools.partial(pl.kernel, mesh=mesh, out_type=out_type,
                       scratch_types=scratch,
                       compiler_params=pltpu.CompilerParams(
                           use_tc_tiling_on_sc=False))
    def seg(*refs):
        if with_deg:
            (table, srch, dsth, out, out_dg, src_v, dst_v,
             *rest) = refs
            gb = rest[:NB]
            zb, ones_v, db, acc, acc_dg = rest[NB:NB + 5]
            gsems = rest[NB + 5:2 * NB + 5]
            ssems = rest[2 * NB + 5:3 * NB + 5]
            dsems = rest[3 * NB + 5:4 * NB + 5]
        else:
            (table, srch, dsth, out, src_v, dst_v, *rest) = refs
            gb = rest[:NB]
            zb, acc = rest[NB:NB + 2]
            gsems = rest[NB + 2:2 * NB + 2]
            ssems = rest[2 * NB + 2:3 * NB + 2]
        c = lax.axis_index("c")
        s = lax.axis_index("s")

        pltpu.sync_copy(srch.at[pl.ds(s * CHT, CHT)], src_v)
        pltpu.sync_copy(dsth.at[pl.ds(s * CHT, CHT)], dst_v)

        _fill(zb, RK, F, 0.0)
        if with_deg:
            _fill(ones_v, CHUNK, 16, 1.0)
            _fill(db, RK, 16, 0.0)

        def zchunk(k, _):
            r0 = s * RT + k * RK
            pltpu.sync_copy(zb, acc.at[pl.ds(r0, RK)])
            return _
        lax.fori_loop(0, RT // RK, zchunk, None)
        pltpu.sync_copy(zb.at[pl.ds(0, RTAIL)],
                        acc.at[pl.ds(s * RT + (RT // RK) * RK, RTAIL)])
        if with_deg:
            def zdchunk(k, _):
                r0 = s * RT + k * RK
                pltpu.sync_copy(db, acc_dg.at[pl.ds(r0, RK)])
                return _
            lax.fori_loop(0, RT // RK, zdchunk, None)
            pltpu.sync_copy(db.at[pl.ds(0, RTAIL)],
                            acc_dg.at[pl.ds(s * RT + (RT // RK) * RK, RTAIL)])
        plsc.subcore_barrier()

        pltpu.async_copy(table.at[c].at[src_v.at[0]], gb[0], gsems[0])
        pltpu.async_copy(table.at[c].at[src_v.at[1]], gb[1], gsems[1])

        def step(j4, _):
            for b in range(NB):
                j = j4 * NB + b
                bp = (b + NB - 1) % NB
                b2 = (b + 2) % NB
                pltpu.make_async_copy(table.at[c].at[src_v.at[j]], gb[b],
                                      gsems[b]).wait()

                @pl.when(j >= 1)
                def _wait_sc():
                    pltpu.make_async_copy(gb[bp], acc.at[dst_v.at[0]],
                                          ssems[bp]).wait()
                    if with_deg:
                        pltpu.make_async_copy(ones_v, acc_dg.at[dst_v.at[0]],
                                              dsems[bp]).wait()
                pltpu.async_copy(gb[b], acc.at[dst_v.at[j]], ssems[b],
                                 add=True)
                if with_deg:
                    @pl.when(c == 0)
                    def _de():
                        pltpu.async_copy(ones_v, acc_dg.at[dst_v.at[j]],
                                         dsems[b], add=True)

                    @pl.when(c == 1)
                    def _dv():
                        pltpu.async_copy(ones_v, acc_dg.at[src_v.at[j]],
                                         dsems[b], add=True)

                @pl.when(j + 2 < CHT)
                def _start():
                    pltpu.async_copy(table.at[c].at[src_v.at[j + 2]], gb[b2],
                                     gsems[b2])
            return _
        lax.fori_loop(0, CHT // NB, step, None)
        pltpu.make_async_copy(gb[(CHT - 1) % NB], acc.at[dst_v.at[0]],
                              ssems[(CHT - 1) % NB]).wait()
        if with_deg:
            pltpu.make_async_copy(ones_v, acc_dg.at[dst_v.at[0]],
                                  dsems[(CHT - 1) % NB]).wait()
        plsc.subcore_barrier()

        def cchunk(k, _):
            r0 = s * RT + k * RK
            pltpu.sync_copy(acc.at[pl.ds(r0, RK)], zb)
            pltpu.sync_copy(zb, out.at[c].at[pl.ds(r0, RK)])
            return _
        lax.fori_loop(0, RT // RK, cchunk, None)
        rt0 = s * RT + (RT // RK) * RK
        pltpu.sync_copy(acc.at[pl.ds(rt0, RTAIL)], zb.at[pl.ds(0, RTAIL)])
        pltpu.sync_copy(zb.at[pl.ds(0, RTAIL)],
                        out.at[c].at[pl.ds(rt0, RTAIL)])
        if with_deg:
            def cdchunk(k, _):
                r0 = s * RT + k * RK
                pltpu.sync_copy(acc_dg.at[pl.ds(r0, RK)], db)
                pltpu.sync_copy(db, out_dg.at[c].at[pl.ds(r0, RK)])
                return _
            lax.fori_loop(0, RT // RK, cdchunk, None)
            pltpu.sync_copy(acc_dg.at[pl.ds(rt0, RTAIL)],
                            db.at[pl.ds(0, RTAIL)])
            pltpu.sync_copy(db.at[pl.ds(0, RTAIL)],
                            out_dg.at[c].at[pl.ds(rt0, RTAIL)])

    return seg


def _make_seg_pair(F):
    CHP = NROWS // (NC * NS)
    mesh = plsc.VectorSubcoreMesh(core_axis_name="c", subcore_axis_name="s")
    out_type = [jax.ShapeDtypeStruct((NC, R_ACC, F), jnp.float32)]
    scratch = [
        pltpu.VMEM((CHP, CHUNK), jnp.int32),
        pltpu.VMEM((CHP, CHUNK), jnp.int32),
    ]
    scratch += [pltpu.VMEM((CHUNK, F), jnp.float32) for _ in range(NB)]
    scratch.append(pltpu.VMEM((RK, F), jnp.float32))
    scratch.append(pltpu.VMEM_SHARED((R_ACC, F), jnp.float32))
    scratch += [pltpu.SemaphoreType.DMA] * (2 * NB)

    @functools.partial(pl.kernel, mesh=mesh, out_type=out_type,
                       scratch_types=scratch,
                       compiler_params=pltpu.CompilerParams(
                           use_tc_tiling_on_sc=False))
    def seg(table, srch, dsth, out, src_v, dst_v, *rest):
        gb = rest[:NB]
        zb, acc = rest[NB:NB + 2]
        gsems = rest[NB + 2:2 * NB + 2]
        ssems = rest[2 * NB + 2:3 * NB + 2]
        c = lax.axis_index("c")
        s = lax.axis_index("s")
        wid = c * NS + s

        pltpu.sync_copy(srch.at[pl.ds(wid * CHP, CHP)], src_v)
        pltpu.sync_copy(dsth.at[pl.ds(wid * CHP, CHP)], dst_v)

        _fill(zb, RK, F, 0.0)

        def zchunk(k, _):
            pltpu.sync_copy(zb, acc.at[pl.ds(s * RT + k * RK, RK)])
            return _
        lax.fori_loop(0, RT // RK, zchunk, None)
        pltpu.sync_copy(zb.at[pl.ds(0, RTAIL)],
                        acc.at[pl.ds(s * RT + (RT // RK) * RK, RTAIL)])
        plsc.subcore_barrier()

        pltpu.async_copy(table.at[src_v.at[0]], gb[0], gsems[0])
        pltpu.async_copy(table.at[src_v.at[1]], gb[1], gsems[1])

        def step(j4, _):
            for b in range(NB):
                j = j4 * NB + b
                bp = (b + NB - 1) % NB
                b2 = (b + 2) % NB
                pltpu.make_async_copy(table.at[src_v.at[j]], gb[b],
                                      gsems[b]).wait()

                @pl.when(j >= 1)
                def _wait_sc():
                    pltpu.make_async_copy(gb[bp], acc.at[dst_v.at[0]],
                                          ssems[bp]).wait()
                pltpu.async_copy(gb[b], acc.at[dst_v.at[j]], ssems[b],
                                 add=True)

                @pl.when(j + 2 < CHP)
                def _start():
                    pltpu.async_copy(table.at[src_v.at[j + 2]], gb[b2],
                                     gsems[b2])
            return _
        lax.fori_loop(0, CHP // NB, step, None)
        pltpu.make_async_copy(gb[(CHP - 1) % NB], acc.at[dst_v.at[0]],
                              ssems[(CHP - 1) % NB]).wait()
        plsc.subcore_barrier()

        def cchunk(k, _):
            r0 = s * RT + k * RK
            pltpu.sync_copy(acc.at[pl.ds(r0, RK)], zb)
            pltpu.sync_copy(zb, out.at[c].at[pl.ds(r0, RK)])
            return _
        lax.fori_loop(0, RT // RK, cchunk, None)
        rt0 = s * RT + (RT // RK) * RK
        pltpu.sync_copy(acc.at[pl.ds(rt0, RTAIL)], zb.at[pl.ds(0, RTAIL)])
        pltpu.sync_copy(zb.at[pl.ds(0, RTAIL)],
                        out.at[c].at[pl.ds(rt0, RTAIL)])

    return seg


_seg64_deg = _make_seg(FH, True)
_seg64 = _make_seg(FH, False)
_seg48 = _make_seg_pair(F_PAD)


def _inv(col):
    return jnp.where(col > 0, 1.0 / col, 0.0)


def _tc_lin1(x, W1, b1r):
    def f(x_ref, w_ref, b_ref, o_ref):
        y = jnp.dot(x_ref[...], w_ref[...],
                    preferred_element_type=jnp.float32,
                    precision=lax.Precision.HIGHEST) + b_ref[...]
        o_ref[0, :N_NODES, :] = y[:, :FH]
        o_ref[1, :N_NODES, :] = y[:, FH:]
        o_ref[0, N_NODES:, :] = jnp.zeros((R_ACC - N_NODES, FH), jnp.float32)
        o_ref[1, N_NODES:, :] = jnp.zeros((R_ACC - N_NODES, FH), jnp.float32)
    return pl.pallas_call(
        f, out_shape=jax.ShapeDtypeStruct((NC, R_ACC, FH), jnp.float32),
    )(x, W1, b1r)


def _make_tc_scale(F, dslot):
    def f(p_ref, d_ref, o_ref):
        inv = _inv(d_ref[dslot, :, 0:1])
        o_ref[0] = inv * p_ref[0]
        o_ref[1] = inv * p_ref[1]
    return pl.pallas_call(
        f, out_shape=jax.ShapeDtypeStruct((NC, R_ACC, F), jnp.float32))


_tc_scale64 = _make_tc_scale(FH, 0)


def _tc_combine48(rp, dg):
    def f(p_ref, d_ref, o_ref):
        inv = _inv(d_ref[0, :, 0:1])
        o_ref[...] = inv * (p_ref[0] + p_ref[1])
    return pl.pallas_call(
        f, out_shape=jax.ShapeDtypeStruct((R_ACC, F_PAD), jnp.float32))(rp, dg)


def _tc_relu_lin2(qp, dg, W2, b2r):
    def f(q_ref, d_ref, w_ref, b_ref, o_ref):
        inv = _inv(d_ref[1, :, 0:1])
        h = inv * jnp.concatenate([q_ref[0], q_ref[1]], axis=1)
        h = jnp.where(h >= 0, h, 0.01 * h)
        y = jnp.dot(h, w_ref[...],
                    preferred_element_type=jnp.float32,
                    precision=lax.Precision.HIGHEST) + b_ref[...]
        o_ref[:N_NODES, :] = y[:N_NODES, :]
        o_ref[N_NODES:, :] = jnp.zeros((R_ACC - N_NODES, F_PAD), jnp.float32)
    return pl.pallas_call(
        f, out_shape=jax.ShapeDtypeStruct((R_ACC, F_PAD), jnp.float32),
    )(qp, dg, W2, b2r)


def _tc_out(sp, dg):
    def f(s_ref, d_ref, o_ref):
        inv = _inv(d_ref[1, :N_NODES, 0:1])
        z = inv * (s_ref[0, :N_NODES, :] + s_ref[1, :N_NODES, :])
        z = z[:, :N_CLASS]
        z = z - jnp.max(z, axis=1, keepdims=True)
        lse = jnp.log(jnp.sum(jnp.exp(z), axis=1, keepdims=True))
        o_ref[...] = z - lse
    return pl.pallas_call(
        f, out_shape=jax.ShapeDtypeStruct((N_NODES, N_CLASS), jnp.float32),
    )(sp, dg)


def kernel(x, H, W1, b1, W2, b2):
    H = H.astype(jnp.int32)
    node = H[0]
    hye = H[1]
    pad = jnp.full((NPP - N_PAIRS,), DUMMY, jnp.int32)
    idx_n = jnp.concatenate([node, pad]).reshape(NROWS, CHUNK)
    idx_e = jnp.concatenate([hye, pad]).reshape(NROWS, CHUNK)
    b1r = b1.reshape(1, HIDDEN)
    W2p = jnp.pad(W2, ((0, 0), (0, F_PAD - N_CLASS)))
    b2r = jnp.pad(b2, (0, F_PAD - N_CLASS)).reshape(1, F_PAD)

    y1 = _tc_lin1(x, W1, b1r)
    ep, dg = _seg64_deg(y1, idx_n, idx_e)
    e1 = _tc_scale64(ep, dg)
    qp, = _seg64(e1, idx_e, idx_n)
    y2 = _tc_relu_lin2(qp, dg, W2p, b2r)
    rp, = _seg48(y2, idx_n, idx_e)
    e2 = _tc_combine48(rp, dg)
    sp, = _seg48(e2, idx_e, idx_n)
    return _tc_out(sp, dg)


# speedup vs baseline: 2.0755x; 2.0755x over previous
import functools

import jax
import jax.numpy as jnp
from jax import lax
from jax.experimental import pallas as pl
from jax.experimental.pallas import tpu as pltpu
from jax.experimental.pallas import tpu_sc as plsc

N_NODES = 10000
N_PAIRS = 320000
HIDDEN = 128
FH = 64
N_CLASS = 40
F_PAD = 48
FC = 24

NC, NS = 2, 16
CHUNK = 128
NB = 4
NPP = 327680
NROWS = NPP // CHUNK
CHT = NROWS // NS
R_ACC = 10240
RT = R_ACC // NS
RK = 32
NDUM = R_ACC - N_NODES


def _fill(ref, rows, width, value):
    def row(i, _):
        for k in range(width // 16):
            ref[i, pl.ds(16 * k, 16)] = jnp.full((16,), value, jnp.float32)
        return _
    lax.fori_loop(0, rows, row, None)


def _make_seg(F, with_deg):
    mesh = plsc.VectorSubcoreMesh(core_axis_name="c", subcore_axis_name="s")
    out_type = [jax.ShapeDtypeStruct((NC, R_ACC, F), jnp.float32)]
    if with_deg:
        out_type.append(jax.ShapeDtypeStruct((NC, R_ACC, 16), jnp.float32))
    scratch = [
        pltpu.VMEM((CHT, CHUNK), jnp.int32),
        pltpu.VMEM((CHT, CHUNK), jnp.int32),
    ]
    scratch += [pltpu.VMEM((CHUNK, F), jnp.float32) for _ in range(NB)]
    scratch.append(pltpu.VMEM((RK, F), jnp.float32))
    if with_deg:
        scratch += [
            pltpu.VMEM((CHUNK, 16), jnp.float32),
            pltpu.VMEM((RK, 16), jnp.float32),
        ]
    scratch.append(pltpu.VMEM_SHARED((R_ACC, F), jnp.float32))
    if with_deg:
        scratch.append(pltpu.VMEM_SHARED((R_ACC, 16), jnp.float32))
    scratch += [pltpu.SemaphoreType.DMA] * (2 * NB)
    if with_deg:
        scratch += [pltpu.SemaphoreType.DMA] * NB

    @functools.partial(pl.kernel, mesh=mesh, out_type=out_type,
                       scratch_types=scratch,
                       compiler_params=pltpu.CompilerParams(
                           use_tc_tiling_on_sc=False))
    def seg(*refs):
        if with_deg:
            (table, srch, dsth, out, out_dg, src_v, dst_v,
             *rest) = refs
            gb = rest[:NB]
            zb, ones_v, db, acc, acc_dg = rest[NB:NB + 5]
            gsems = rest[NB + 5:2 * NB + 5]
            ssems = rest[2 * NB + 5:3 * NB + 5]
            dsems = rest[3 * NB + 5:4 * NB + 5]
        else:
            (table, srch, dsth, out, src_v, dst_v, *rest) = refs
            gb = rest[:NB]
            zb, acc = rest[NB:NB + 2]
            gsems = rest[NB + 2:2 * NB + 2]
            ssems = rest[2 * NB + 2:3 * NB + 2]
        c = lax.axis_index("c")
        s = lax.axis_index("s")

        pltpu.sync_copy(srch.at[pl.ds(s * CHT, CHT)], src_v)
        pltpu.sync_copy(dsth.at[pl.ds(s * CHT, CHT)], dst_v)

        _fill(zb, RK, F, 0.0)
        if with_deg:
            _fill(ones_v, CHUNK, 16, 1.0)
            _fill(db, RK, 16, 0.0)

        def zchunk(k, _):
            r0 = s * RT + k * RK
            pltpu.sync_copy(zb, acc.at[pl.ds(r0, RK)])
            return _
        lax.fori_loop(0, RT // RK, zchunk, None)
        if with_deg:
            def zdchunk(k, _):
                r0 = s * RT + k * RK
                pltpu.sync_copy(db, acc_dg.at[pl.ds(r0, RK)])
                return _
            lax.fori_loop(0, RT // RK, zdchunk, None)
        plsc.subcore_barrier()

        pltpu.async_copy(table.at[c].at[src_v.at[0]], gb[0], gsems[0])
        pltpu.async_copy(table.at[c].at[src_v.at[1]], gb[1], gsems[1])

        def step(j4, _):
            for b in range(NB):
                j = j4 * NB + b
                bp = (b + NB - 1) % NB
                b2 = (b + 2) % NB
                pltpu.make_async_copy(table.at[c].at[src_v.at[j]], gb[b],
                                      gsems[b]).wait()

                @pl.when(j >= 1)
                def _wait_sc():
                    pltpu.make_async_copy(gb[bp], acc.at[dst_v.at[0]],
                                          ssems[bp]).wait()
                    if with_deg:
                        pltpu.make_async_copy(ones_v, acc_dg.at[dst_v.at[0]],
                                              dsems[bp]).wait()
                pltpu.async_copy(gb[b], acc.at[dst_v.at[j]], ssems[b],
                                 add=True)
                if with_deg:
                    @pl.when(c == 0)
                    def _de():
                        pltpu.async_copy(ones_v, acc_dg.at[dst_v.at[j]],
                                         dsems[b], add=True)

                    @pl.when(c == 1)
                    def _dv():
                        pltpu.async_copy(ones_v, acc_dg.at[src_v.at[j]],
                                         dsems[b], add=True)

                @pl.when(j + 2 < CHT)
                def _start():
                    pltpu.async_copy(table.at[c].at[src_v.at[j + 2]], gb[b2],
                                     gsems[b2])
            return _
        lax.fori_loop(0, CHT // NB, step, None)
        pltpu.make_async_copy(gb[(CHT - 1) % NB], acc.at[dst_v.at[0]],
                              ssems[(CHT - 1) % NB]).wait()
        if with_deg:
            pltpu.make_async_copy(ones_v, acc_dg.at[dst_v.at[0]],
                                  dsems[(CHT - 1) % NB]).wait()
        plsc.subcore_barrier()

        def cchunk(k, _):
            r0 = s * RT + k * RK
            pltpu.sync_copy(acc.at[pl.ds(r0, RK)], zb)
            pltpu.sync_copy(zb, out.at[c].at[pl.ds(r0, RK)])
            return _
        lax.fori_loop(0, RT // RK, cchunk, None)
        if with_deg:
            def cdchunk(k, _):
                r0 = s * RT + k * RK
                pltpu.sync_copy(acc_dg.at[pl.ds(r0, RK)], db)
                pltpu.sync_copy(db, out_dg.at[c].at[pl.ds(r0, RK)])
                return _
            lax.fori_loop(0, RT // RK, cdchunk, None)

    return seg


def _make_seg_pair(F):
    CHP = NROWS // (NC * NS)
    mesh = plsc.VectorSubcoreMesh(core_axis_name="c", subcore_axis_name="s")
    out_type = [jax.ShapeDtypeStruct((NC, R_ACC, F), jnp.float32)]
    scratch = [
        pltpu.VMEM((CHP, CHUNK), jnp.int32),
        pltpu.VMEM((CHP, CHUNK), jnp.int32),
    ]
    scratch += [pltpu.VMEM((CHUNK, F), jnp.float32) for _ in range(NB)]
    scratch.append(pltpu.VMEM((RK, F), jnp.float32))
    scratch.append(pltpu.VMEM_SHARED((R_ACC, F), jnp.float32))
    scratch += [pltpu.SemaphoreType.DMA] * (2 * NB)

    @functools.partial(pl.kernel, mesh=mesh, out_type=out_type,
                       scratch_types=scratch,
                       compiler_params=pltpu.CompilerParams(
                           use_tc_tiling_on_sc=False))
    def seg(table, srch, dsth, out, src_v, dst_v, *rest):
        gb = rest[:NB]
        zb, acc = rest[NB:NB + 2]
        gsems = rest[NB + 2:2 * NB + 2]
        ssems = rest[2 * NB + 2:3 * NB + 2]
        c = lax.axis_index("c")
        s = lax.axis_index("s")
        wid = c * NS + s

        pltpu.sync_copy(srch.at[pl.ds(wid * CHP, CHP)], src_v)
        pltpu.sync_copy(dsth.at[pl.ds(wid * CHP, CHP)], dst_v)

        _fill(zb, RK, F, 0.0)

        def zchunk(k, _):
            pltpu.sync_copy(zb, acc.at[pl.ds(s * RT + k * RK, RK)])
            return _
        lax.fori_loop(0, RT // RK, zchunk, None)
        plsc.subcore_barrier()

        pltpu.async_copy(table.at[src_v.at[0]], gb[0], gsems[0])
        pltpu.async_copy(table.at[src_v.at[1]], gb[1], gsems[1])

        def step(j4, _):
            for b in range(NB):
                j = j4 * NB + b
                bp = (b + NB - 1) % NB
                b2 = (b + 2) % NB
                pltpu.make_async_copy(table.at[src_v.at[j]], gb[b],
                                      gsems[b]).wait()

                @pl.when(j >= 1)
                def _wait_sc():
                    pltpu.make_async_copy(gb[bp], acc.at[dst_v.at[0]],
                                          ssems[bp]).wait()
                pltpu.async_copy(gb[b], acc.at[dst_v.at[j]], ssems[b],
                                 add=True)

                @pl.when(j + 2 < CHP)
                def _start():
                    pltpu.async_copy(table.at[src_v.at[j + 2]], gb[b2],
                                     gsems[b2])
            return _
        lax.fori_loop(0, CHP // NB, step, None)
        pltpu.make_async_copy(gb[(CHP - 1) % NB], acc.at[dst_v.at[0]],
                              ssems[(CHP - 1) % NB]).wait()
        plsc.subcore_barrier()

        def cchunk(k, _):
            r0 = s * RT + k * RK
            pltpu.sync_copy(acc.at[pl.ds(r0, RK)], zb)
            pltpu.sync_copy(zb, out.at[c].at[pl.ds(r0, RK)])
            return _
        lax.fori_loop(0, RT // RK, cchunk, None)

    return seg


_seg64_deg = _make_seg(FH, True)
_seg64 = _make_seg(FH, False)
_seg48 = _make_seg_pair(F_PAD)


def _inv(col):
    return jnp.where(col > 0, 1.0 / col, 0.0)


def _tc_lin1(x, W1, b1r):
    def f(x_ref, w_ref, b_ref, o_ref):
        y = jnp.dot(x_ref[...], w_ref[...],
                    preferred_element_type=jnp.float32,
                    precision=lax.Precision.HIGHEST) + b_ref[...]
        o_ref[0, :N_NODES, :] = y[:, :FH]
        o_ref[1, :N_NODES, :] = y[:, FH:]
        o_ref[0, N_NODES:, :] = jnp.zeros((R_ACC - N_NODES, FH), jnp.float32)
        o_ref[1, N_NODES:, :] = jnp.zeros((R_ACC - N_NODES, FH), jnp.float32)
    return pl.pallas_call(
        f, out_shape=jax.ShapeDtypeStruct((NC, R_ACC, FH), jnp.float32),
    )(x, W1, b1r)


def _make_tc_scale(F, dslot):
    def f(p_ref, d_ref, o_ref):
        inv = _inv(d_ref[dslot, :, 0:1])
        o_ref[0] = inv * p_ref[0]
        o_ref[1] = inv * p_ref[1]
    return pl.pallas_call(
        f, out_shape=jax.ShapeDtypeStruct((NC, R_ACC, F), jnp.float32))


_tc_scale64 = _make_tc_scale(FH, 0)


def _tc_combine48(rp, dg):
    def f(p_ref, d_ref, o_ref):
        inv = _inv(d_ref[0, :, 0:1])
        o_ref[...] = inv * (p_ref[0] + p_ref[1])
    return pl.pallas_call(
        f, out_shape=jax.ShapeDtypeStruct((R_ACC, F_PAD), jnp.float32))(rp, dg)


def _tc_relu_lin2(qp, dg, W2, b2r):
    def f(q_ref, d_ref, w_ref, b_ref, o_ref):
        inv = _inv(d_ref[1, :, 0:1])
        h = inv * jnp.concatenate([q_ref[0], q_ref[1]], axis=1)
        h = jnp.where(h >= 0, h, 0.01 * h)
        y = jnp.dot(h, w_ref[...],
                    preferred_element_type=jnp.float32,
                    precision=lax.Precision.HIGHEST) + b_ref[...]
        o_ref[:N_NODES, :] = y[:N_NODES, :]
        o_ref[N_NODES:, :] = jnp.zeros((R_ACC - N_NODES, F_PAD), jnp.float32)
    return pl.pallas_call(
        f, out_shape=jax.ShapeDtypeStruct((R_ACC, F_PAD), jnp.float32),
    )(qp, dg, W2, b2r)


def _tc_out(sp, dg):
    def f(s_ref, d_ref, o_ref):
        inv = _inv(d_ref[1, :N_NODES, 0:1])
        z = inv * (s_ref[0, :N_NODES, :] + s_ref[1, :N_NODES, :])
        z = z[:, :N_CLASS]
        z = z - jnp.max(z, axis=1, keepdims=True)
        lse = jnp.log(jnp.sum(jnp.exp(z), axis=1, keepdims=True))
        o_ref[...] = z - lse
    return pl.pallas_call(
        f, out_shape=jax.ShapeDtypeStruct((N_NODES, N_CLASS), jnp.float32),
    )(sp, dg)


def kernel(x, H, W1, b1, W2, b2):
    H = H.astype(jnp.int32)
    node = H[0]
    hye = H[1]
    pad = N_NODES + (jnp.arange(NPP - N_PAIRS, dtype=jnp.int32) % NDUM)
    idx_n = jnp.concatenate([node, pad]).reshape(NROWS, CHUNK)
    idx_e = jnp.concatenate([hye, pad]).reshape(NROWS, CHUNK)
    b1r = b1.reshape(1, HIDDEN)
    W2p = jnp.pad(W2, ((0, 0), (0, F_PAD - N_CLASS)))
    b2r = jnp.pad(b2, (0, F_PAD - N_CLASS)).reshape(1, F_PAD)

    y1 = _tc_lin1(x, W1, b1r)
    ep, dg = _seg64_deg(y1, idx_n, idx_e)
    e1 = _tc_scale64(ep, dg)
    qp, = _seg64(e1, idx_e, idx_n)
    y2 = _tc_relu_lin2(qp, dg, W2p, b2r)
    rp, = _seg48(y2, idx_n, idx_e)
    e2 = _tc_combine48(rp, dg)
    sp, = _seg48(e2, idx_e, idx_n)
    return _tc_out(sp, dg)


# speedup vs baseline: 2.1802x; 1.0504x over previous
import functools

import jax
import jax.numpy as jnp
from jax import lax
from jax.experimental import pallas as pl
from jax.experimental.pallas import tpu as pltpu
from jax.experimental.pallas import tpu_sc as plsc

N_NODES = 10000
N_PAIRS = 320000
HIDDEN = 128
FH = 64
N_CLASS = 40
F_PAD = 48
FC = 24

NC, NS = 2, 16
CHUNK = 128
NB = 4
NPP = 327680
NROWS = NPP // CHUNK
CHT = NROWS // NS
R_ACC = 10240
RT = R_ACC // NS
RK = 32
NDUM = R_ACC - N_NODES


def _fill(ref, rows, width, value):
    def row(i, _):
        for k in range(width // 16):
            ref[i, pl.ds(16 * k, 16)] = jnp.full((16,), value, jnp.float32)
        return _
    lax.fori_loop(0, rows, row, None)


def _make_seg(F, with_deg):
    mesh = plsc.VectorSubcoreMesh(core_axis_name="c", subcore_axis_name="s")
    out_type = [jax.ShapeDtypeStruct((NC, R_ACC, F), jnp.float32)]
    if with_deg:
        out_type.append(jax.ShapeDtypeStruct((NC, R_ACC, 16), jnp.float32))
    scratch = [
        pltpu.VMEM((CHT, CHUNK), jnp.int32),
        pltpu.VMEM((CHT, CHUNK), jnp.int32),
    ]
    scratch += [pltpu.VMEM((CHUNK, F), jnp.float32) for _ in range(NB)]
    scratch.append(pltpu.VMEM((RK, F), jnp.float32))
    if with_deg:
        scratch += [
            pltpu.VMEM((CHUNK, 16), jnp.float32),
            pltpu.VMEM((RK, 16), jnp.float32),
        ]
    scratch.append(pltpu.VMEM_SHARED((R_ACC, F), jnp.float32))
    if with_deg:
        scratch.append(pltpu.VMEM_SHARED((R_ACC, 16), jnp.float32))
    scratch += [pltpu.SemaphoreType.DMA] * (2 * NB)
    if with_deg:
        scratch += [pltpu.SemaphoreType.DMA] * NB

    @functools.partial(pl.kernel, mesh=mesh, out_type=out_type,
                       scratch_types=scratch,
                       compiler_params=pltpu.CompilerParams(
                           use_tc_tiling_on_sc=False))
    def seg(*refs):
        if with_deg:
            (table, srch, dsth, out, out_dg, src_v, dst_v,
             *rest) = refs
            gb = rest[:NB]
            zb, ones_v, db, acc, acc_dg = rest[NB:NB + 5]
            gsems = rest[NB + 5:2 * NB + 5]
            ssems = rest[2 * NB + 5:3 * NB + 5]
            dsems = rest[3 * NB + 5:4 * NB + 5]
        else:
            (table, srch, dsth, out, src_v, dst_v, *rest) = refs
            gb = rest[:NB]
            zb, acc = rest[NB:NB + 2]
            gsems = rest[NB + 2:2 * NB + 2]
            ssems = rest[2 * NB + 2:3 * NB + 2]
        c = lax.axis_index("c")
        s = lax.axis_index("s")

        pltpu.async_copy(srch.at[pl.ds(s * CHT, CHT)], src_v, gsems[0])
        pltpu.async_copy(dsth.at[pl.ds(s * CHT, CHT)], dst_v, gsems[1])

        _fill(zb, RK, F, 0.0)
        if with_deg:
            _fill(ones_v, CHUNK, 16, 1.0)
            _fill(db, RK, 16, 0.0)

        def zchunk(k4, _):
            for b in range(NB):
                r0 = s * RT + (k4 * NB + b) * RK
                pltpu.async_copy(zb, acc.at[pl.ds(r0, RK)], ssems[b])
                if with_deg:
                    pltpu.async_copy(db, acc_dg.at[pl.ds(r0, RK)], dsems[b])
            return _
        lax.fori_loop(0, RT // RK // NB, zchunk, None)

        def zdrain(k4, _):
            for b in range(NB):
                r0 = s * RT + (k4 * NB + b) * RK
                pltpu.make_async_copy(zb, acc.at[pl.ds(r0, RK)],
                                      ssems[b]).wait()
                if with_deg:
                    pltpu.make_async_copy(db, acc_dg.at[pl.ds(r0, RK)],
                                          dsems[b]).wait()
            return _
        lax.fori_loop(0, RT // RK // NB, zdrain, None)
        pltpu.make_async_copy(srch.at[pl.ds(s * CHT, CHT)], src_v,
                              gsems[0]).wait()
        pltpu.make_async_copy(dsth.at[pl.ds(s * CHT, CHT)], dst_v,
                              gsems[1]).wait()
        plsc.subcore_barrier()

        pltpu.async_copy(table.at[c].at[src_v.at[0]], gb[0], gsems[0])
        pltpu.async_copy(table.at[c].at[src_v.at[1]], gb[1], gsems[1])

        def step(j4, _):
            for b in range(NB):
                j = j4 * NB + b
                bp = (b + NB - 1) % NB
                b2 = (b + 2) % NB
                pltpu.make_async_copy(table.at[c].at[src_v.at[j]], gb[b],
                                      gsems[b]).wait()

                @pl.when(j >= 1)
                def _wait_sc():
                    pltpu.make_async_copy(gb[bp], acc.at[dst_v.at[0]],
                                          ssems[bp]).wait()
                    if with_deg:
                        pltpu.make_async_copy(ones_v, acc_dg.at[dst_v.at[0]],
                                              dsems[bp]).wait()
                pltpu.async_copy(gb[b], acc.at[dst_v.at[j]], ssems[b],
                                 add=True)
                if with_deg:
                    @pl.when(c == 0)
                    def _de():
                        pltpu.async_copy(ones_v, acc_dg.at[dst_v.at[j]],
                                         dsems[b], add=True)

                    @pl.when(c == 1)
                    def _dv():
                        pltpu.async_copy(ones_v, acc_dg.at[src_v.at[j]],
                                         dsems[b], add=True)

                @pl.when(j + 2 < CHT)
                def _start():
                    pltpu.async_copy(table.at[c].at[src_v.at[j + 2]], gb[b2],
                                     gsems[b2])
            return _
        lax.fori_loop(0, CHT // NB, step, None)
        pltpu.make_async_copy(gb[(CHT - 1) % NB], acc.at[dst_v.at[0]],
                              ssems[(CHT - 1) % NB]).wait()
        if with_deg:
            pltpu.make_async_copy(ones_v, acc_dg.at[dst_v.at[0]],
                                  dsems[(CHT - 1) % NB]).wait()
        plsc.subcore_barrier()

        def cchunk(k4, _):
            for b in range(NB):
                k = k4 * NB + b
                r0 = s * RT + k * RK

                @pl.when(k >= NB)
                def _wait_wr():
                    pltpu.make_async_copy(gb[b].at[pl.ds(0, RK)],
                                          out.at[c].at[pl.ds(0, RK)],
                                          gsems[b]).wait()
                pltpu.sync_copy(acc.at[pl.ds(r0, RK)], gb[b].at[pl.ds(0, RK)])
                pltpu.async_copy(gb[b].at[pl.ds(0, RK)],
                                 out.at[c].at[pl.ds(r0, RK)], gsems[b])
            return _
        lax.fori_loop(0, RT // RK // NB, cchunk, None)
        for b in range(NB):
            pltpu.make_async_copy(gb[b].at[pl.ds(0, RK)],
                                  out.at[c].at[pl.ds(0, RK)], gsems[b]).wait()
        if with_deg:
            def cdchunk(k, _):
                r0 = s * RT + k * RK
                pltpu.sync_copy(acc_dg.at[pl.ds(r0, RK)], db)
                pltpu.sync_copy(db, out_dg.at[c].at[pl.ds(r0, RK)])
                return _
            lax.fori_loop(0, RT // RK, cdchunk, None)

    return seg


def _make_seg_pair(F):
    CHP = NROWS // (NC * NS)
    mesh = plsc.VectorSubcoreMesh(core_axis_name="c", subcore_axis_name="s")
    out_type = [jax.ShapeDtypeStruct((NC, R_ACC, F), jnp.float32)]
    scratch = [
        pltpu.VMEM((CHP, CHUNK), jnp.int32),
        pltpu.VMEM((CHP, CHUNK), jnp.int32),
    ]
    scratch += [pltpu.VMEM((CHUNK, F), jnp.float32) for _ in range(NB)]
    scratch.append(pltpu.VMEM((RK, F), jnp.float32))
    scratch.append(pltpu.VMEM_SHARED((R_ACC, F), jnp.float32))
    scratch += [pltpu.SemaphoreType.DMA] * (2 * NB)

    @functools.partial(pl.kernel, mesh=mesh, out_type=out_type,
                       scratch_types=scratch,
                       compiler_params=pltpu.CompilerParams(
                           use_tc_tiling_on_sc=False))
    def seg(table, srch, dsth, out, src_v, dst_v, *rest):
        gb = rest[:NB]
        zb, acc = rest[NB:NB + 2]
        gsems = rest[NB + 2:2 * NB + 2]
        ssems = rest[2 * NB + 2:3 * NB + 2]
        c = lax.axis_index("c")
        s = lax.axis_index("s")
        wid = c * NS + s

        pltpu.async_copy(srch.at[pl.ds(wid * CHP, CHP)], src_v, gsems[0])
        pltpu.async_copy(dsth.at[pl.ds(wid * CHP, CHP)], dst_v, gsems[1])

        _fill(zb, RK, F, 0.0)

        def zchunk(k4, _):
            for b in range(NB):
                r0 = s * RT + (k4 * NB + b) * RK
                pltpu.async_copy(zb, acc.at[pl.ds(r0, RK)], ssems[b])
            return _
        lax.fori_loop(0, RT // RK // NB, zchunk, None)

        def zdrain(k4, _):
            for b in range(NB):
                r0 = s * RT + (k4 * NB + b) * RK
                pltpu.make_async_copy(zb, acc.at[pl.ds(r0, RK)],
                                      ssems[b]).wait()
            return _
        lax.fori_loop(0, RT // RK // NB, zdrain, None)
        pltpu.make_async_copy(srch.at[pl.ds(wid * CHP, CHP)], src_v,
                              gsems[0]).wait()
        pltpu.make_async_copy(dsth.at[pl.ds(wid * CHP, CHP)], dst_v,
                              gsems[1]).wait()
        plsc.subcore_barrier()

        pltpu.async_copy(table.at[src_v.at[0]], gb[0], gsems[0])
        pltpu.async_copy(table.at[src_v.at[1]], gb[1], gsems[1])

        def step(j4, _):
            for b in range(NB):
                j = j4 * NB + b
                bp = (b + NB - 1) % NB
                b2 = (b + 2) % NB
                pltpu.make_async_copy(table.at[src_v.at[j]], gb[b],
                                      gsems[b]).wait()

                @pl.when(j >= 1)
                def _wait_sc():
                    pltpu.make_async_copy(gb[bp], acc.at[dst_v.at[0]],
                                          ssems[bp]).wait()
                pltpu.async_copy(gb[b], acc.at[dst_v.at[j]], ssems[b],
                                 add=True)

                @pl.when(j + 2 < CHP)
                def _start():
                    pltpu.async_copy(table.at[src_v.at[j + 2]], gb[b2],
                                     gsems[b2])
            return _
        lax.fori_loop(0, CHP // NB, step, None)
        pltpu.make_async_copy(gb[(CHP - 1) % NB], acc.at[dst_v.at[0]],
                              ssems[(CHP - 1) % NB]).wait()
        plsc.subcore_barrier()

        def cchunk(k4, _):
            for b in range(NB):
                k = k4 * NB + b
                r0 = s * RT + k * RK

                @pl.when(k >= NB)
                def _wait_wr():
                    pltpu.make_async_copy(gb[b].at[pl.ds(0, RK)],
                                          out.at[c].at[pl.ds(0, RK)],
                                          gsems[b]).wait()
                pltpu.sync_copy(acc.at[pl.ds(r0, RK)], gb[b].at[pl.ds(0, RK)])
                pltpu.async_copy(gb[b].at[pl.ds(0, RK)],
                                 out.at[c].at[pl.ds(r0, RK)], gsems[b])
            return _
        lax.fori_loop(0, RT // RK // NB, cchunk, None)
        for b in range(NB):
            pltpu.make_async_copy(gb[b].at[pl.ds(0, RK)],
                                  out.at[c].at[pl.ds(0, RK)], gsems[b]).wait()

    return seg


_seg64_deg = _make_seg(FH, True)
_seg64 = _make_seg(FH, False)
_seg48 = _make_seg_pair(F_PAD)


def _inv(col):
    return jnp.where(col > 0, 1.0 / col, 0.0)


def _tc_lin1(x, W1, b1r):
    def f(x_ref, w_ref, b_ref, o_ref):
        y = jnp.dot(x_ref[...], w_ref[...],
                    preferred_element_type=jnp.float32,
                    precision=lax.Precision.HIGHEST) + b_ref[...]
        o_ref[0, :N_NODES, :] = y[:, :FH]
        o_ref[1, :N_NODES, :] = y[:, FH:]
        o_ref[0, N_NODES:, :] = jnp.zeros((R_ACC - N_NODES, FH), jnp.float32)
        o_ref[1, N_NODES:, :] = jnp.zeros((R_ACC - N_NODES, FH), jnp.float32)
    return pl.pallas_call(
        f, out_shape=jax.ShapeDtypeStruct((NC, R_ACC, FH), jnp.float32),
    )(x, W1, b1r)


def _make_tc_scale(F, dslot):
    def f(p_ref, d_ref, o_ref):
        inv = _inv(d_ref[dslot, :, 0:1])
        o_ref[0] = inv * p_ref[0]
        o_ref[1] = inv * p_ref[1]
    return pl.pallas_call(
        f, out_shape=jax.ShapeDtypeStruct((NC, R_ACC, F), jnp.float32))


_tc_scale64 = _make_tc_scale(FH, 0)


def _tc_combine48(rp, dg):
    def f(p_ref, d_ref, o_ref):
        inv = _inv(d_ref[0, :, 0:1])
        o_ref[...] = inv * (p_ref[0] + p_ref[1])
    return pl.pallas_call(
        f, out_shape=jax.ShapeDtypeStruct((R_ACC, F_PAD), jnp.float32))(rp, dg)


def _tc_relu_lin2(qp, dg, W2, b2r):
    def f(q_ref, d_ref, w_ref, b_ref, o_ref):
        inv = _inv(d_ref[1, :, 0:1])
        h = inv * jnp.concatenate([q_ref[0], q_ref[1]], axis=1)
        h = jnp.where(h >= 0, h, 0.01 * h)
        y = jnp.dot(h, w_ref[...],
                    preferred_element_type=jnp.float32,
                    precision=lax.Precision.HIGHEST) + b_ref[...]
        o_ref[:N_NODES, :] = y[:N_NODES, :]
        o_ref[N_NODES:, :] = jnp.zeros((R_ACC - N_NODES, F_PAD), jnp.float32)
    return pl.pallas_call(
        f, out_shape=jax.ShapeDtypeStruct((R_ACC, F_PAD), jnp.float32),
    )(qp, dg, W2, b2r)


def _tc_out(sp, dg):
    def f(s_ref, d_ref, o_ref):
        inv = _inv(d_ref[1, :N_NODES, 0:1])
        z = inv * (s_ref[0, :N_NODES, :] + s_ref[1, :N_NODES, :])
        z = z[:, :N_CLASS]
        z = z - jnp.max(z, axis=1, keepdims=True)
        lse = jnp.log(jnp.sum(jnp.exp(z), axis=1, keepdims=True))
        o_ref[...] = z - lse
    return pl.pallas_call(
        f, out_shape=jax.ShapeDtypeStruct((N_NODES, N_CLASS), jnp.float32),
    )(sp, dg)


def kernel(x, H, W1, b1, W2, b2):
    H = H.astype(jnp.int32)
    node = H[0]
    hye = H[1]
    pad = N_NODES + (jnp.arange(NPP - N_PAIRS, dtype=jnp.int32) % NDUM)
    idx_n = jnp.concatenate([node, pad]).reshape(NROWS, CHUNK)
    idx_e = jnp.concatenate([hye, pad]).reshape(NROWS, CHUNK)
    b1r = b1.reshape(1, HIDDEN)
    W2p = jnp.pad(W2, ((0, 0), (0, F_PAD - N_CLASS)))
    b2r = jnp.pad(b2, (0, F_PAD - N_CLASS)).reshape(1, F_PAD)

    y1 = _tc_lin1(x, W1, b1r)
    ep, dg = _seg64_deg(y1, idx_n, idx_e)
    e1 = _tc_scale64(ep, dg)
    qp, = _seg64(e1, idx_e, idx_n)
    y2 = _tc_relu_lin2(qp, dg, W2p, b2r)
    rp, = _seg48(y2, idx_n, idx_e)
    e2 = _tc_combine48(rp, dg)
    sp, = _seg48(e2, idx_e, idx_n)
    return _tc_out(sp, dg)


# speedup vs baseline: 2.3163x; 1.0624x over previous
import functools

import jax
import jax.numpy as jnp
from jax import lax
from jax.experimental import pallas as pl
from jax.experimental.pallas import tpu as pltpu
from jax.experimental.pallas import tpu_sc as plsc

N_NODES = 10000
N_PAIRS = 320000
HIDDEN = 128
FH = 64
N_CLASS = 40
F_PAD = 48
FC = 24

NC, NS = 2, 16
CHUNK = 128
NB = 4
NPP = 327680
NROWS = NPP // CHUNK
CHT = NROWS // NS
R_ACC = 10240
RT = R_ACC // NS
RK = 32
NDUM = R_ACC - N_NODES


def _fill(ref, rows, width, value):
    def row(i, _):
        for k in range(width // 16):
            ref[i, pl.ds(16 * k, 16)] = jnp.full((16,), value, jnp.float32)
        return _
    lax.fori_loop(0, rows, row, None)


def _make_seg(F, with_deg):
    mesh = plsc.VectorSubcoreMesh(core_axis_name="c", subcore_axis_name="s")
    out_type = [jax.ShapeDtypeStruct((NC, R_ACC, F), jnp.float32)]
    if with_deg:
        out_type.append(jax.ShapeDtypeStruct((NC, R_ACC, 16), jnp.float32))
    scratch = [
        pltpu.VMEM((CHT, CHUNK), jnp.int32),
        pltpu.VMEM((CHT, CHUNK), jnp.int32),
    ]
    scratch += [pltpu.VMEM((CHUNK, F), jnp.float32) for _ in range(NB)]
    scratch.append(pltpu.VMEM((RK, F), jnp.float32))
    if with_deg:
        scratch += [
            pltpu.VMEM((CHUNK, 16), jnp.float32),
            pltpu.VMEM((RK, 16), jnp.float32),
        ]
    scratch.append(pltpu.VMEM_SHARED((R_ACC, F), jnp.float32))
    if with_deg:
        scratch.append(pltpu.VMEM_SHARED((R_ACC, 16), jnp.float32))
    scratch += [pltpu.SemaphoreType.DMA] * (2 * NB)
    if with_deg:
        scratch += [pltpu.SemaphoreType.DMA] * NB

    @functools.partial(pl.kernel, mesh=mesh, out_type=out_type,
                       scratch_types=scratch,
                       compiler_params=pltpu.CompilerParams(
                           use_tc_tiling_on_sc=False))
    def seg(*refs):
        if with_deg:
            (table, srch, dsth, out, out_dg, src_v, dst_v,
             *rest) = refs
            gb = rest[:NB]
            zb, ones_v, db, acc, acc_dg = rest[NB:NB + 5]
            gsems = rest[NB + 5:2 * NB + 5]
            ssems = rest[2 * NB + 5:3 * NB + 5]
            dsems = rest[3 * NB + 5:4 * NB + 5]
        else:
            (table, srch, dsth, out, src_v, dst_v, *rest) = refs
            gb = rest[:NB]
            zb, acc = rest[NB:NB + 2]
            gsems = rest[NB + 2:2 * NB + 2]
            ssems = rest[2 * NB + 2:3 * NB + 2]
        c = lax.axis_index("c")
        s = lax.axis_index("s")

        pltpu.async_copy(srch.at[pl.ds(s * CHT, CHT)], src_v, gsems[0])
        pltpu.async_copy(dsth.at[pl.ds(s * CHT, CHT)], dst_v, gsems[1])

        _fill(zb, RK, F, 0.0)
        if with_deg:
            _fill(ones_v, CHUNK, 16, 1.0)
            _fill(db, RK, 16, 0.0)

        def zchunk(k4, _):
            for b in range(NB):
                r0 = s * RT + (k4 * NB + b) * RK
                pltpu.async_copy(zb, acc.at[pl.ds(r0, RK)], ssems[b])
                if with_deg:
                    pltpu.async_copy(db, acc_dg.at[pl.ds(r0, RK)], dsems[b])
            return _
        lax.fori_loop(0, RT // RK // NB, zchunk, None)

        def zdrain(k4, _):
            for b in range(NB):
                r0 = s * RT + (k4 * NB + b) * RK
                pltpu.make_async_copy(zb, acc.at[pl.ds(r0, RK)],
                                      ssems[b]).wait()
                if with_deg:
                    pltpu.make_async_copy(db, acc_dg.at[pl.ds(r0, RK)],
                                          dsems[b]).wait()
            return _
        lax.fori_loop(0, RT // RK // NB, zdrain, None)
        pltpu.make_async_copy(srch.at[pl.ds(s * CHT, CHT)], src_v,
                              gsems[0]).wait()
        pltpu.make_async_copy(dsth.at[pl.ds(s * CHT, CHT)], dst_v,
                              gsems[1]).wait()
        pltpu.async_copy(table.at[c].at[src_v.at[0]], gb[0], gsems[0])
        pltpu.async_copy(table.at[c].at[src_v.at[1]], gb[1], gsems[1])
        pltpu.async_copy(table.at[c].at[src_v.at[2]], gb[2], gsems[2])
        plsc.subcore_barrier()

        def step(j4, _):
            for b in range(NB):
                j = j4 * NB + b
                bp = (b + NB - 1) % NB
                pltpu.make_async_copy(table.at[c].at[src_v.at[j]], gb[b],
                                      gsems[b]).wait()

                @pl.when(j >= 1)
                def _wait_sc():
                    pltpu.make_async_copy(gb[bp], acc.at[dst_v.at[0]],
                                          ssems[bp]).wait()
                    if with_deg:
                        pltpu.make_async_copy(ones_v, acc_dg.at[dst_v.at[0]],
                                              dsems[bp]).wait()
                pltpu.async_copy(gb[b], acc.at[dst_v.at[j]], ssems[b],
                                 add=True)
                if with_deg:
                    @pl.when(c == 0)
                    def _de():
                        pltpu.async_copy(ones_v, acc_dg.at[dst_v.at[j]],
                                         dsems[b], add=True)

                    @pl.when(c == 1)
                    def _dv():
                        pltpu.async_copy(ones_v, acc_dg.at[src_v.at[j]],
                                         dsems[b], add=True)

                @pl.when(j + 3 < CHT)
                def _start():
                    pltpu.async_copy(table.at[c].at[src_v.at[j + 3]], gb[bp],
                                     gsems[bp])
            return _
        lax.fori_loop(0, CHT // NB, step, None)
        pltpu.make_async_copy(gb[(CHT - 1) % NB], acc.at[dst_v.at[0]],
                              ssems[(CHT - 1) % NB]).wait()
        if with_deg:
            pltpu.make_async_copy(ones_v, acc_dg.at[dst_v.at[0]],
                                  dsems[(CHT - 1) % NB]).wait()
        plsc.subcore_barrier()

        def cchunk(k4, _):
            for b in range(NB):
                k = k4 * NB + b
                r0 = s * RT + k * RK

                @pl.when(k >= NB)
                def _wait_wr():
                    pltpu.make_async_copy(gb[b].at[pl.ds(0, RK)],
                                          out.at[c].at[pl.ds(0, RK)],
                                          gsems[b]).wait()
                pltpu.sync_copy(acc.at[pl.ds(r0, RK)], gb[b].at[pl.ds(0, RK)])
                pltpu.async_copy(gb[b].at[pl.ds(0, RK)],
                                 out.at[c].at[pl.ds(r0, RK)], gsems[b])
            return _
        lax.fori_loop(0, RT // RK // NB, cchunk, None)
        for b in range(NB):
            pltpu.make_async_copy(gb[b].at[pl.ds(0, RK)],
                                  out.at[c].at[pl.ds(0, RK)], gsems[b]).wait()
        if with_deg:
            def cdchunk(k, _):
                r0 = s * RT + k * RK
                pltpu.sync_copy(acc_dg.at[pl.ds(r0, RK)], db)
                pltpu.sync_copy(db, out_dg.at[c].at[pl.ds(r0, RK)])
                return _
            lax.fori_loop(0, RT // RK, cdchunk, None)

    return seg


def _make_seg_pair(F):
    CHP = NROWS // (NC * NS)
    mesh = plsc.VectorSubcoreMesh(core_axis_name="c", subcore_axis_name="s")
    out_type = [jax.ShapeDtypeStruct((NC, R_ACC, F), jnp.float32)]
    scratch = [
        pltpu.VMEM((CHP, CHUNK), jnp.int32),
        pltpu.VMEM((CHP, CHUNK), jnp.int32),
    ]
    scratch += [pltpu.VMEM((CHUNK, F), jnp.float32) for _ in range(NB)]
    scratch.append(pltpu.VMEM((RK, F), jnp.float32))
    scratch.append(pltpu.VMEM_SHARED((R_ACC, F), jnp.float32))
    scratch += [pltpu.SemaphoreType.DMA] * (2 * NB)

    @functools.partial(pl.kernel, mesh=mesh, out_type=out_type,
                       scratch_types=scratch,
                       compiler_params=pltpu.CompilerParams(
                           use_tc_tiling_on_sc=False))
    def seg(table, srch, dsth, out, src_v, dst_v, *rest):
        gb = rest[:NB]
        zb, acc = rest[NB:NB + 2]
        gsems = rest[NB + 2:2 * NB + 2]
        ssems = rest[2 * NB + 2:3 * NB + 2]
        c = lax.axis_index("c")
        s = lax.axis_index("s")
        wid = c * NS + s

        pltpu.async_copy(srch.at[pl.ds(wid * CHP, CHP)], src_v, gsems[0])
        pltpu.async_copy(dsth.at[pl.ds(wid * CHP, CHP)], dst_v, gsems[1])

        _fill(zb, RK, F, 0.0)

        def zchunk(k4, _):
            for b in range(NB):
                r0 = s * RT + (k4 * NB + b) * RK
                pltpu.async_copy(zb, acc.at[pl.ds(r0, RK)], ssems[b])
            return _
        lax.fori_loop(0, RT // RK // NB, zchunk, None)

        def zdrain(k4, _):
            for b in range(NB):
                r0 = s * RT + (k4 * NB + b) * RK
                pltpu.make_async_copy(zb, acc.at[pl.ds(r0, RK)],
                                      ssems[b]).wait()
            return _
        lax.fori_loop(0, RT // RK // NB, zdrain, None)
        pltpu.make_async_copy(srch.at[pl.ds(wid * CHP, CHP)], src_v,
                              gsems[0]).wait()
        pltpu.make_async_copy(dsth.at[pl.ds(wid * CHP, CHP)], dst_v,
                              gsems[1]).wait()
        pltpu.async_copy(table.at[src_v.at[0]], gb[0], gsems[0])
        pltpu.async_copy(table.at[src_v.at[1]], gb[1], gsems[1])
        pltpu.async_copy(table.at[src_v.at[2]], gb[2], gsems[2])
        plsc.subcore_barrier()

        def step(j4, _):
            for b in range(NB):
                j = j4 * NB + b
                bp = (b + NB - 1) % NB
                pltpu.make_async_copy(table.at[src_v.at[j]], gb[b],
                                      gsems[b]).wait()

                @pl.when(j >= 1)
                def _wait_sc():
                    pltpu.make_async_copy(gb[bp], acc.at[dst_v.at[0]],
                                          ssems[bp]).wait()
                pltpu.async_copy(gb[b], acc.at[dst_v.at[j]], ssems[b],
                                 add=True)

                @pl.when(j + 3 < CHP)
                def _start():
                    pltpu.async_copy(table.at[src_v.at[j + 3]], gb[bp],
                                     gsems[bp])
            return _
        lax.fori_loop(0, CHP // NB, step, None)
        pltpu.make_async_copy(gb[(CHP - 1) % NB], acc.at[dst_v.at[0]],
                              ssems[(CHP - 1) % NB]).wait()
        plsc.subcore_barrier()

        def cchunk(k4, _):
            for b in range(NB):
                k = k4 * NB + b
                r0 = s * RT + k * RK

                @pl.when(k >= NB)
                def _wait_wr():
                    pltpu.make_async_copy(gb[b].at[pl.ds(0, RK)],
                                          out.at[c].at[pl.ds(0, RK)],
                                          gsems[b]).wait()
                pltpu.sync_copy(acc.at[pl.ds(r0, RK)], gb[b].at[pl.ds(0, RK)])
                pltpu.async_copy(gb[b].at[pl.ds(0, RK)],
                                 out.at[c].at[pl.ds(r0, RK)], gsems[b])
            return _
        lax.fori_loop(0, RT // RK // NB, cchunk, None)
        for b in range(NB):
            pltpu.make_async_copy(gb[b].at[pl.ds(0, RK)],
                                  out.at[c].at[pl.ds(0, RK)], gsems[b]).wait()

    return seg


_seg64_deg = _make_seg(FH, True)
_seg64 = _make_seg(FH, False)
_seg48 = _make_seg_pair(F_PAD)


def _inv(col):
    return jnp.where(col > 0, 1.0 / col, 0.0)


def _tc_lin1(x, W1, b1r):
    def f(x_ref, w_ref, b_ref, o_ref):
        y = jnp.dot(x_ref[...], w_ref[...],
                    preferred_element_type=jnp.float32,
                    precision=lax.Precision.HIGHEST) + b_ref[...]
        o_ref[0, :N_NODES, :] = y[:, :FH]
        o_ref[1, :N_NODES, :] = y[:, FH:]
        o_ref[0, N_NODES:, :] = jnp.zeros((R_ACC - N_NODES, FH), jnp.float32)
        o_ref[1, N_NODES:, :] = jnp.zeros((R_ACC - N_NODES, FH), jnp.float32)
    return pl.pallas_call(
        f, out_shape=jax.ShapeDtypeStruct((NC, R_ACC, FH), jnp.float32),
    )(x, W1, b1r)


def _make_tc_scale(F, dslot):
    def f(p_ref, d_ref, o_ref):
        inv = _inv(d_ref[dslot, :, 0:1])
        o_ref[0] = inv * p_ref[0]
        o_ref[1] = inv * p_ref[1]
    return pl.pallas_call(
        f, out_shape=jax.ShapeDtypeStruct((NC, R_ACC, F), jnp.float32))


_tc_scale64 = _make_tc_scale(FH, 0)


def _tc_combine48(rp, dg):
    def f(p_ref, d_ref, o_ref):
        inv = _inv(d_ref[0, :, 0:1])
        o_ref[...] = inv * (p_ref[0] + p_ref[1])
    return pl.pallas_call(
        f, out_shape=jax.ShapeDtypeStruct((R_ACC, F_PAD), jnp.float32))(rp, dg)


def _tc_relu_lin2(qp, dg, W2, b2r):
    def f(q_ref, d_ref, w_ref, b_ref, o_ref):
        inv = _inv(d_ref[1, :, 0:1])
        h = inv * jnp.concatenate([q_ref[0], q_ref[1]], axis=1)
        h = jnp.where(h >= 0, h, 0.01 * h)
        y = jnp.dot(h, w_ref[...],
                    preferred_element_type=jnp.float32,
                    precision=lax.Precision.HIGHEST) + b_ref[...]
        o_ref[:N_NODES, :] = y[:N_NODES, :]
        o_ref[N_NODES:, :] = jnp.zeros((R_ACC - N_NODES, F_PAD), jnp.float32)
    return pl.pallas_call(
        f, out_shape=jax.ShapeDtypeStruct((R_ACC, F_PAD), jnp.float32),
    )(qp, dg, W2, b2r)


def _tc_out(sp, dg):
    def f(s_ref, d_ref, o_ref):
        inv = _inv(d_ref[1, :N_NODES, 0:1])
        z = inv * (s_ref[0, :N_NODES, :] + s_ref[1, :N_NODES, :])
        z = z[:, :N_CLASS]
        z = z - jnp.max(z, axis=1, keepdims=True)
        lse = jnp.log(jnp.sum(jnp.exp(z), axis=1, keepdims=True))
        o_ref[...] = z - lse
    return pl.pallas_call(
        f, out_shape=jax.ShapeDtypeStruct((N_NODES, N_CLASS), jnp.float32),
    )(sp, dg)


def kernel(x, H, W1, b1, W2, b2):
    H = H.astype(jnp.int32)
    node = H[0]
    hye = H[1]
    pad = N_NODES + (jnp.arange(NPP - N_PAIRS, dtype=jnp.int32) % NDUM)
    idx_n = jnp.concatenate([node, pad]).reshape(NROWS, CHUNK)
    idx_e = jnp.concatenate([hye, pad]).reshape(NROWS, CHUNK)
    b1r = b1.reshape(1, HIDDEN)
    W2p = jnp.pad(W2, ((0, 0), (0, F_PAD - N_CLASS)))
    b2r = jnp.pad(b2, (0, F_PAD - N_CLASS)).reshape(1, F_PAD)

    y1 = _tc_lin1(x, W1, b1r)
    ep, dg = _seg64_deg(y1, idx_n, idx_e)
    e1 = _tc_scale64(ep, dg)
    qp, = _seg64(e1, idx_e, idx_n)
    y2 = _tc_relu_lin2(qp, dg, W2p, b2r)
    rp, = _seg48(y2, idx_n, idx_e)
    e2 = _tc_combine48(rp, dg)
    sp, = _seg48(e2, idx_e, idx_n)
    return _tc_out(sp, dg)
